# baseline (speedup 1.0000x reference)
import functools

import jax
import jax.numpy as jnp
from jax import lax
from jax.experimental import pallas as pl
from jax.experimental.pallas import tpu as pltpu
from jax.experimental.pallas import tpu_sc as plsc

D = 128
S = 10
NC = 2
NS = 16
NW = NC * NS
BPAD = 51200
BPW = BPAD // NW
C = 80
NCHUNK = BPW // C
NBUF = 4
NG = NCHUNK // NBUF
MM_BLK = 1024


@functools.cache
def _make_sc_gather_sum():
    mesh = plsc.VectorSubcoreMesh(core_axis_name="c", subcore_axis_name="s")

    @functools.partial(
        pl.kernel,
        out_type=(
            jax.ShapeDtypeStruct((BPAD, D), jnp.float32),
            jax.ShapeDtypeStruct((BPAD, D), jnp.float32),
        ),
        mesh=mesh,
        scratch_types=[
            pltpu.VMEM((BPW,), jnp.int32),
            pltpu.VMEM((S * BPW,), jnp.int32),
            pltpu.VMEM((NBUF, C, D), jnp.float32),
            pltpu.VMEM((NBUF, C, D), jnp.float32),
        ] + [pltpu.SemaphoreType.DMA] * (2 * NBUF),
    )
    def _sc_gather_sum(x_hbm, nodes_hbm, neight_hbm, self_out, sum_out,
                       idx_s, idx_n, rows_s, acc_v, *sems):
        _sc_body(x_hbm, nodes_hbm, neight_hbm, self_out, sum_out,
                 idx_s, idx_n, rows_s, acc_v,
                 sems[:NBUF], sems[NBUF:])

    return _sc_gather_sum


def _sc_body(x_hbm, nodes_hbm, neight_hbm, self_out, sum_out,
             idx_s, idx_n, rows_s, acc_v, gsems, ssems):
    wid = lax.axis_index("s") * NC + lax.axis_index("c")
    base = wid * BPW

    pltpu.sync_copy(nodes_hbm.at[pl.ds(base, BPW)], idx_s)
    for j in range(S):
        pltpu.sync_copy(neight_hbm.at[pl.ds(j * BPAD + base, BPW)],
                        idx_n.at[pl.ds(j * BPW, BPW)])

    zeros16 = jnp.zeros((16,), jnp.float32)

    def zero_acc(b):
        def zbody(r, carry):
            for k in range(D // 16):
                acc_v[b, r, pl.ds(k * 16, 16)] = zeros16
            return carry
        lax.fori_loop(0, C, zbody, 0, unroll=False)

    def fire(ci, b):
        c0 = ci * C
        pltpu.async_copy(x_hbm.at[idx_s.at[pl.ds(c0, C)]], rows_s.at[b],
                         gsems[b])
        for j in range(S):
            pltpu.async_copy(x_hbm.at[idx_n.at[pl.ds(j * BPW + c0, C)]],
                             acc_v.at[b], gsems[b], add=True)

    def drain_gathers(ci, b):
        c0 = ci * C
        pltpu.make_async_copy(x_hbm.at[idx_s.at[pl.ds(c0, C)]], rows_s.at[b],
                              gsems[b]).wait()
        for j in range(S):
            pltpu.make_async_copy(x_hbm.at[idx_n.at[pl.ds(j * BPW + c0, C)]],
                                  acc_v.at[b], gsems[b]).wait()

    def store(ci, b):
        row0 = base + ci * C
        pltpu.async_copy(rows_s.at[b], self_out.at[pl.ds(row0, C)], ssems[b])
        pltpu.async_copy(acc_v.at[b], sum_out.at[pl.ds(row0, C)], ssems[b])

    def drain_store(b):
        pltpu.make_async_copy(rows_s.at[b], self_out.at[pl.ds(base, C)],
                              ssems[b]).wait()
        pltpu.make_async_copy(acc_v.at[b], sum_out.at[pl.ds(base, C)],
                              ssems[b]).wait()

    for b in range(NBUF):
        zero_acc(b)
        fire(b, b)

    def ring_body(g, carry):
        for b in range(NBUF):
            ci = NBUF * g + b
            drain_gathers(ci, b)
            store(ci, b)

            @pl.when(g < NG - 1)
            def _():
                drain_store(b)
                zero_acc(b)
                fire(ci + NBUF, b)
        return carry

    lax.fori_loop(0, NG, ring_body, 0, unroll=False)
    for b in range(NBUF):
        drain_store(b)


def _mm_body(self_ref, sum_ref, w1_ref, w2_ref, o_ref):
    a = lax.dot_general(w1_ref[...], self_ref[...],
                        (((1,), (1,)), ((), ())),
                        preferred_element_type=jnp.float32)
    b = lax.dot_general(w2_ref[...], sum_ref[...],
                        (((1,), (1,)), ((), ())),
                        preferred_element_type=jnp.float32)
    o_ref[...] = jnp.maximum(a + b, 0.0)


_tc_matmul = pl.pallas_call(
    _mm_body,
    grid=(BPAD // MM_BLK,),
    in_specs=[
        pl.BlockSpec((MM_BLK, D), lambda i: (i, 0)),
        pl.BlockSpec((MM_BLK, D), lambda i: (i, 0)),
        pl.BlockSpec((D, D), lambda i: (0, 0)),
        pl.BlockSpec((D, D), lambda i: (0, 0)),
    ],
    out_specs=pl.BlockSpec((D, MM_BLK), lambda i: (0, i)),
    out_shape=jax.ShapeDtypeStruct((D, BPAD), jnp.float32),
)


def kernel(x, W, nodes, neigh_idx):
    B = nodes.shape[0]
    pad = BPAD - B
    nodes_p = jnp.concatenate([nodes, jnp.zeros((pad,), jnp.int32)])
    neigh_t = jnp.concatenate(
        [neigh_idx, jnp.zeros((pad, S), jnp.int32)]).T.reshape(-1)
    self_feats, neigh_sum = _make_sc_gather_sum()(x, nodes_p, neigh_t)
    w1 = W[:, :D]
    w2 = W[:, D:] * jnp.float32(1.0 / S)
    out = _tc_matmul(self_feats, neigh_sum, w1, w2)
    return out[:, :B]


# speedup vs baseline: 3.0941x; 3.0941x over previous
import functools

import jax
import jax.numpy as jnp
from jax import lax
from jax.experimental import pallas as pl
from jax.experimental.pallas import tpu as pltpu
from jax.experimental.pallas import tpu_sc as plsc

D = 128
S = 10
NC = 2
NS = 16
NW = NC * NS
BPAD = 50176
BPW = BPAD // NW
C = 56
NCHUNK = BPW // C
NBUF = 2
NG = NCHUNK // NBUF
MM_BLK = 1024


@functools.cache
def _make_sc_gather_sum():
    mesh = plsc.VectorSubcoreMesh(core_axis_name="c", subcore_axis_name="s")

    @functools.partial(
        pl.kernel,
        out_type=(
            jax.ShapeDtypeStruct((BPAD, D), jnp.float32),
            jax.ShapeDtypeStruct((BPAD, D), jnp.float32),
        ),
        mesh=mesh,
        scratch_types=[
            pltpu.VMEM((BPW,), jnp.int32),
            pltpu.VMEM((S * BPW,), jnp.int32),
            pltpu.VMEM((NBUF, C, D), jnp.float32),
            pltpu.VMEM((NBUF, C, D), jnp.float32),
        ] + [pltpu.SemaphoreType.DMA] * (2 * NBUF),
    )
    def _sc_gather_sum(x_hbm, nodes_hbm, neigh_hbm, self_out, sum_out,
                       idx_s, idx_n, rows_s, acc_v, *sems):
        _sc_body(x_hbm, nodes_hbm, neigh_hbm, self_out, sum_out,
                 idx_s, idx_n, rows_s, acc_v,
                 sems[:NBUF], sems[NBUF:])

    return _sc_gather_sum


def _sc_body(x_hbm, nodes_hbm, neigh_hbm, self_out, sum_out,
             idx_s, idx_n, rows_s, acc_v, gsems, ssems):
    wid = lax.axis_index("s") * NC + lax.axis_index("c")
    base = wid * BPW

    pltpu.sync_copy(nodes_hbm.at[pl.ds(base, BPW)], idx_s)
    for j in range(S):
        pltpu.sync_copy(neigh_hbm.at[pl.ds(j * BPAD + base, BPW)],
                        idx_n.at[pl.ds(j * BPW, BPW)])

    zeros16 = jnp.zeros((16,), jnp.float32)

    def zero_acc(b):
        def zbody(r, carry):
            for k in range(D // 16):
                acc_v[b, r, pl.ds(k * 16, 16)] = zeros16
            return carry
        lax.fori_loop(0, C, zbody, 0, unroll=False)

    def fire(ci, b):
        c0 = ci * C
        pltpu.async_copy(x_hbm.at[idx_s.at[pl.ds(c0, C)]], rows_s.at[b],
                         gsems[b])
        for j in range(S):
            pltpu.async_copy(x_hbm.at[idx_n.at[pl.ds(j * BPW + c0, C)]],
                             acc_v.at[b], gsems[b], add=True)

    def drain_gathers(ci, b):
        c0 = ci * C
        pltpu.make_async_copy(x_hbm.at[idx_s.at[pl.ds(c0, C)]], rows_s.at[b],
                              gsems[b]).wait()
        for j in range(S):
            pltpu.make_async_copy(x_hbm.at[idx_n.at[pl.ds(j * BPW + c0, C)]],
                                  acc_v.at[b], gsems[b]).wait()

    def store(ci, b):
        row0 = base + ci * C
        pltpu.async_copy(rows_s.at[b], self_out.at[pl.ds(row0, C)], ssems[b])
        pltpu.async_copy(acc_v.at[b], sum_out.at[pl.ds(row0, C)], ssems[b])

    def drain_store(b):
        pltpu.make_async_copy(rows_s.at[b], self_out.at[pl.ds(base, C)],
                              ssems[b]).wait()
        pltpu.make_async_copy(acc_v.at[b], sum_out.at[pl.ds(base, C)],
                              ssems[b]).wait()

    for b in range(NBUF):
        zero_acc(b)
        fire(b, b)

    def ring_body(g, carry):
        for b in range(NBUF):
            ci = NBUF * g + b
            drain_gathers(ci, b)
            store(ci, b)

            @pl.when(g < NG - 1)
            def _():
                drain_store(b)
                zero_acc(b)
                fire(ci + NBUF, b)
        return carry

    lax.fori_loop(0, NG, ring_body, 0, unroll=False)
    for b in range(NBUF):
        drain_store(b)


def _mm_body(self_ref, sum_ref, w1_ref, w2_ref, o_ref):
    a = lax.dot_general(w1_ref[...], self_ref[...],
                        (((1,), (1,)), ((), ())),
                        preferred_element_type=jnp.float32)
    b = lax.dot_general(w2_ref[...], sum_ref[...],
                        (((1,), (1,)), ((), ())),
                        preferred_element_type=jnp.float32)
    o_ref[...] = jnp.maximum(a + b, 0.0)


def _make_tc_matmul(B):
    return pl.pallas_call(
        _mm_body,
        grid=(BPAD // MM_BLK,),
        in_specs=[
            pl.BlockSpec((MM_BLK, D), lambda i: (i, 0)),
            pl.BlockSpec((MM_BLK, D), lambda i: (i, 0)),
            pl.BlockSpec((D, D), lambda i: (0, 0)),
            pl.BlockSpec((D, D), lambda i: (0, 0)),
        ],
        out_specs=pl.BlockSpec((D, MM_BLK), lambda i: (0, i)),
        out_shape=jax.ShapeDtypeStruct((D, B), jnp.float32),
    )


def kernel(x, W, nodes, neigh_idx):
    B = nodes.shape[0]
    pad = BPAD - B
    nodes_p = jnp.concatenate([nodes, jnp.zeros((pad,), jnp.int32)])
    neigh_f = jnp.concatenate(
        [neigh_idx, jnp.zeros((pad, S), jnp.int32)]).T.reshape(-1)
    self_feats, neigh_sum = _make_sc_gather_sum()(x, nodes_p, neigh_f)
    w1 = W[:, :D]
    w2 = W[:, D:] * jnp.float32(1.0 / S)
    return _make_tc_matmul(B)(self_feats, neigh_sum, w1, w2)
